# baseline (speedup 1.0000x reference)
import functools

import jax
import jax.numpy as jnp
from jax import lax
from jax.experimental import pallas as pl
from jax.experimental.pallas import tpu as pltpu
from jax.experimental.pallas import tpu_sc as plsc


_NC, _NS, _L = 2, 16, 16
_NW = _NC * _NS
_NCHK = 10
_UN = 5


def _sc_kernel(B, V, logits_hbm, actions_hbm, ea_hbm, s_hbm, mode_hbm,
               probs_hbm, xbuf, abuf, c0buf, c1buf, eabuf, sbuf, modebuf,
               asmem, in_sem, out_sem):
    RPW = B // _NW
    C = V // _NCHK
    NV = C // _L
    wid = lax.axis_index("s") * _NC + lax.axis_index("c")
    base = wid * RPW
    lane = lax.iota(jnp.int32, _L)
    lane0 = lane == 0
    ninf = jnp.full((_L,), -jnp.inf, jnp.float32)
    zero_i = jnp.zeros((_L,), jnp.int32)
    zero_f = jnp.zeros((_L,), jnp.float32)

    pltpu.sync_copy(actions_hbm.at[pl.ds(base, RPW)], abuf)
    for gg in range(RPW // _L):
        av = abuf[pl.ds(gg * _L, _L)]
        for t in range(_L):
            asmem[gg * _L + t] = av[t]

    def in_copy(r, k):
        return pltpu.make_async_copy(
            logits_hbm.at[base + r, pl.ds(k * C, C)],
            xbuf.at[pl.ds(k * C, C)],
            in_sem.at[k],
        )

    def out_copy(r, k, cbuf, sem_slot):
        return pltpu.make_async_copy(
            cbuf,
            probs_hbm.at[base + r, pl.ds(k * C, C)],
            out_sem.at[sem_slot],
        )

    for k in range(_NCHK):
        in_copy(0, k).start()

    def row_body(r, carry):
        def scan_chunk(k, acc):
            in_copy(r, k).wait()
            cb = k * C

            def scan_vec(it, acc):
                vms = acc[:_UN]
                vis = acc[_UN:2 * _UN]
                ss = acc[2 * _UN:]
                new = []
                Jb = k * NV + it * _UN
                for u in range(_UN):
                    off = cb + (it * _UN + u) * _L
                    v = xbuf[pl.ds(off, _L)]
                    e = jnp.exp(v)
                    xbuf[pl.ds(off, _L)] = e
                    gt = v > vms[u]
                    new.append((jnp.maximum(v, vms[u]),
                                jnp.where(gt, Jb + u, vis[u]),
                                ss[u] + e))
                return (tuple(x[0] for x in new) + tuple(x[1] for x in new)
                        + tuple(x[2] for x in new))

            return lax.fori_loop(0, NV // _UN, scan_vec, acc)

        acc0 = (ninf,) * _UN + (zero_i,) * _UN + (zero_f,) * _UN
        acc = lax.fori_loop(0, _NCHK, scan_chunk, acc0)
        vms = acc[:_UN]
        vis = acc[_UN:2 * _UN]
        ss = acc[2 * _UN:]

        m = jnp.max(vms[0])
        for u in range(1, _UN):
            m = jnp.maximum(m, jnp.max(vms[u]))
        idx = jnp.int32(V)
        for u in range(_UN):
            cand = jnp.where(vms[u] == m, vis[u] * _L + lane, jnp.int32(V))
            idx = jnp.minimum(idx, jnp.min(cand))
        s = jnp.sum(ss[0])
        for u in range(1, _UN):
            s = s + jnp.sum(ss[u])

        a = asmem[r]
        ea = xbuf[pl.ds(a, _L)][0]

        plsc.store_scatter(eabuf, [jnp.full((_L,), r, jnp.int32)],
                           jnp.full((_L,), ea), mask=lane0)
        plsc.store_scatter(sbuf, [jnp.full((_L,), r, jnp.int32)],
                           jnp.full((_L,), s), mask=lane0)
        plsc.store_scatter(modebuf, [jnp.full((_L,), r, jnp.int32)],
                           jnp.full((_L,), idx), mask=lane0)

        sv = jnp.full((_L,), s, jnp.float32)
        rv = jnp.full((_L,), 100.0, jnp.float32) / sv

        def scale_chunk(k, carry):
            cb = k * C

            @pl.when(jnp.logical_or(r > 0, k >= 2))
            def _wait_prev_out():
                pk = k - 2 + jnp.where(k >= 2, 0, _NCHK)
                pr = r - jnp.where(k >= 2, 0, 1)
                sl = lax.rem(k, 2)

                @pl.when(sl == 0)
                def _w0():
                    out_copy(pr, pk, c0buf, 0).wait()

                @pl.when(sl == 1)
                def _w1():
                    out_copy(pr, pk, c1buf, 1).wait()

            def do_scale(cbuf):
                def sv_body(it, c):
                    for u in range(_UN):
                        off = (it * _UN + u) * _L
                        cbuf[pl.ds(off, _L)] = xbuf[pl.ds(cb + off, _L)] * rv
                    return c
                lax.fori_loop(0, NV // _UN, sv_body, 0)

            @pl.when(lax.rem(k, 2) == 0)
            def _s0():
                do_scale(c0buf)
                out_copy(r, k, c0buf, 0).start()

            @pl.when(lax.rem(k, 2) == 1)
            def _s1():
                do_scale(c1buf)
                out_copy(r, k, c1buf, 1).start()

            @pl.when(r < RPW - 1)
            def _refill():
                in_copy(r + 1, k).start()

            return carry

        lax.fori_loop(0, _NCHK, scale_chunk, 0)
        return carry

    lax.fori_loop(0, RPW, row_body, 0)

    out_copy(RPW - 1, _NCHK - 2, c0buf, 0).wait()
    out_copy(RPW - 1, _NCHK - 1, c1buf, 1).wait()

    pltpu.sync_copy(eabuf, ea_hbm.at[pl.ds(base, RPW)])
    pltpu.sync_copy(sbuf, s_hbm.at[pl.ds(base, RPW)])
    pltpu.sync_copy(modebuf, mode_hbm.at[pl.ds(base, RPW)])


def _lp_kernel(ea_ref, s_ref, lp_ref):
    lp_ref[...] = jnp.log(ea_ref[...] / s_ref[...])


def kernel(logits, actions):
    B, V = logits.shape
    RPW = B // _NW
    C = V // _NCHK
    mesh = plsc.VectorSubcoreMesh(core_axis_name="c", subcore_axis_name="s")
    sc = functools.partial(
        pl.kernel,
        out_type=[
            jax.ShapeDtypeStruct((B,), jnp.float32),
            jax.ShapeDtypeStruct((B,), jnp.float32),
            jax.ShapeDtypeStruct((B,), jnp.int32),
            jax.ShapeDtypeStruct((B, V), jnp.float32),
        ],
        mesh=mesh,
        compiler_params=pltpu.CompilerParams(
            needs_layout_passes=False, use_tc_tiling_on_sc=False),
        scratch_types=[
            pltpu.VMEM((V + _L,), jnp.float32),
            pltpu.VMEM((RPW,), jnp.int32),
            pltpu.VMEM((C,), jnp.float32),
            pltpu.VMEM((C,), jnp.float32),
            pltpu.VMEM((RPW,), jnp.float32),
            pltpu.VMEM((RPW,), jnp.float32),
            pltpu.VMEM((RPW,), jnp.int32),
            pltpu.SMEM((RPW,), jnp.int32),
            pltpu.SemaphoreType.DMA((_NCHK,)),
            pltpu.SemaphoreType.DMA((2,)),
        ],
    )(functools.partial(_sc_kernel, B, V))
    ea, s, mode, new_probs = sc(logits, actions.reshape(B))

    lp = pl.pallas_call(
        _lp_kernel,
        out_shape=jax.ShapeDtypeStruct((8, B // 8), jnp.float32),
    )(ea.reshape(8, B // 8), s.reshape(8, B // 8))

    return (lp.reshape(B, 1), mode.reshape(B, 1), new_probs)
